# baseline (speedup 1.0000x reference)
import functools
import math

import jax
import jax.numpy as jnp
from jax import lax
from jax.experimental import pallas as pl
from jax.experimental.pallas import tpu as pltpu
from jax.experimental.pallas import tpu_sc as plsc

D_MODEL = 64
SCALE = math.sqrt(D_MODEL)

_NC = 2
_NS = 16
_NW = _NC * _NS
_CH = 128
_NIN = 4
_NOUT = 2


def _make_kernel(B: int):
    assert B % (_NW * _CH * _NIN) == 0
    n_chunks = B // (_NW * _CH)
    b_per_w = n_chunks * _CH
    mesh = plsc.VectorSubcoreMesh(core_axis_name="c", subcore_axis_name="s")

    @functools.partial(
        pl.kernel,
        out_type=jax.ShapeDtypeStruct((B, D_MODEL), jnp.float32),
        mesh=mesh,
        scratch_types=[
            pltpu.VMEM((n_chunks, _CH), jnp.int32),
            [pltpu.VMEM((_CH, D_MODEL), jnp.float32) for _ in range(_NIN)],
            [pltpu.VMEM((_CH, D_MODEL), jnp.float32) for _ in range(_NOUT)],
            pltpu.SemaphoreType.DMA((_NIN,)),
            pltpu.SemaphoreType.DMA((_NOUT,)),
        ],
        compiler_params=pltpu.CompilerParams(use_tc_tiling_on_sc=False),
    )
    def embed(idx_hbm, table_hbm, out_hbm, idx_v, ins, outs, gsem, osem):
        wid = lax.axis_index("s") * _NC + lax.axis_index("c")
        base = wid * b_per_w
        pltpu.sync_copy(idx_hbm.at[wid], idx_v)

        def start_gather(c, b):
            pltpu.async_copy(table_hbm.at[idx_v.at[c]], ins[b], gsem.at[b])

        def out_slice(c):
            return out_hbm.at[pl.ds(base + c * _CH, _CH)]

        for b in range(_NIN):
            start_gather(b, b)

        @pl.loop(0, n_chunks, step=_NIN)
        def round_body(r):
            for b in range(_NIN):
                c = r + b
                bo = b % _NOUT
                pltpu.make_async_copy(table_hbm.at[idx_v.at[c]], ins[b],
                                      gsem.at[b]).wait()
                if b >= _NOUT:
                    pltpu.make_async_copy(outs[bo], out_slice(c), osem.at[bo]).wait()
                else:
                    @pl.when(r > 0)
                    def _():
                        pltpu.make_async_copy(outs[bo], out_slice(c),
                                              osem.at[bo]).wait()

                def row_body(rr, _):
                    for cc in range(D_MODEL // 16):
                        sl = pl.ds(cc * 16, 16)
                        outs[bo][rr, sl] = ins[b][rr, sl] * SCALE
                    return ()
                lax.fori_loop(0, _CH, row_body, (), unroll=4)

                pltpu.async_copy(outs[bo], out_slice(c), osem.at[bo])
                nxt = c + _NIN
                @pl.when(nxt < n_chunks)
                def _():
                    start_gather(nxt, b)

        for bo in range(_NOUT):
            c = n_chunks - _NOUT + bo
            pltpu.make_async_copy(outs[bo], out_slice(c), osem.at[bo]).wait()

    return embed


@jax.jit
def kernel(x, table):
    orig_shape = x.shape
    B = x.size
    idx = x.reshape(_NW, B // (_NW * _CH), _CH).astype(jnp.int32)
    out = _make_kernel(B)(idx, table)
    return out.reshape(*orig_shape, D_MODEL)


# speedup vs baseline: 1.2673x; 1.2673x over previous
import functools
import math

import jax
import jax.numpy as jnp
from jax import lax
from jax.experimental import pallas as pl
from jax.experimental.pallas import tpu as pltpu
from jax.experimental.pallas import tpu_sc as plsc

D_MODEL = 64
SCALE = math.sqrt(D_MODEL)

_NC = 2
_NS = 16
_NW = _NC * _NS
_CH = 128
_K = 2
_NSLOT = 4


def _make_kernel(B: int):
    n_chunks = B // (_NW * _CH)
    n_groups = n_chunks // _K
    assert B % (_NW * _CH * _K) == 0 and n_groups % _NSLOT == 0
    mesh = plsc.VectorSubcoreMesh(core_axis_name="c", subcore_axis_name="s")

    @functools.partial(
        pl.kernel,
        out_type=jax.ShapeDtypeStruct((_NW * n_chunks, _CH, D_MODEL),
                                      jnp.float32),
        mesh=mesh,
        scratch_types=[
            pltpu.VMEM((n_chunks, _CH), jnp.int32),
            [pltpu.VMEM((_K, _CH, D_MODEL), jnp.float32)
             for _ in range(_NSLOT)],
            pltpu.SemaphoreType.DMA((_NSLOT * _K,)),
            pltpu.SemaphoreType.DMA((_NSLOT,)),
        ],
        compiler_params=pltpu.CompilerParams(use_tc_tiling_on_sc=False),
    )
    def embed(idx_hbm, table_hbm, out_hbm, idx_v, slots, gsem, osem):
        wid = lax.axis_index("s") * _NC + lax.axis_index("c")
        cbase = wid * n_chunks
        pltpu.sync_copy(idx_hbm.at[wid], idx_v)

        def fire_group(g, s):
            for k in range(_K):
                pltpu.async_copy(table_hbm.at[idx_v.at[g * _K + k]],
                                 slots[s].at[k], gsem.at[s * _K + k])

        def wo_copy(s, g):
            return pltpu.make_async_copy(
                slots[s], out_hbm.at[pl.ds(cbase + g * _K, _K)], osem.at[s])

        fire_group(0, 0)
        fire_group(1, 1)

        @pl.loop(0, n_groups, step=_NSLOT)
        def body(r):
            for j in range(_NSLOT):
                t = r + j
                s = j
                s2 = (j + 2) % _NSLOT

                @pl.when(t >= 2)
                def _():
                    wo_copy(s2, t - 2).wait()

                @pl.when(t + 2 < n_groups)
                def _():
                    fire_group(t + 2, s2)

                for k in range(_K):
                    pltpu.make_async_copy(
                        table_hbm.at[idx_v.at[t * _K + k]],
                        slots[s].at[k], gsem.at[s * _K + k]).wait()

                def row_body(rr, _):
                    for k in range(_K):
                        for c in range(D_MODEL // 16):
                            sl = pl.ds(c * 16, 16)
                            slots[s][k, rr, sl] = slots[s][k, rr, sl] * SCALE
                    return ()
                lax.fori_loop(0, _CH, row_body, (), unroll=8)

                pltpu.async_copy(
                    slots[s], out_hbm.at[pl.ds(cbase + t * _K, _K)],
                    osem.at[s])

        for t in (n_groups - 2, n_groups - 1):
            wo_copy(t % _NSLOT, t).wait()

    return embed


@jax.jit
def kernel(x, table):
    orig_shape = x.shape
    B = x.size
    idx = x.reshape(_NW, B // (_NW * _CH), _CH).astype(jnp.int32)
    out = _make_kernel(B)(idx, table)
    return out.reshape(*orig_shape, D_MODEL)
